# baseline (speedup 1.0000x reference)
import functools

import jax
import jax.numpy as jnp
from jax import lax
from jax.experimental import pallas as pl
from jax.experimental.pallas import tpu as pltpu
from jax.experimental.pallas import tpu_sc as plsc

_NC = 2
_NS = 16
_NW = _NC * _NS
_B = 16384
_D = 32
_R = 1_000_000
_SEC = 32768
_SUP = 1024
_NSUP = _SEC // _SUP
_LAST_FULL = 975
_TAIL_BASE = 976 * _SUP
_TAIL = 640
_RING = 8

_mesh = plsc.VectorSubcoreMesh(core_axis_name="c", subcore_axis_name="s")


@functools.partial(
    pl.kernel,
    mesh=_mesh,
    out_type=(
        jax.ShapeDtypeStruct((_B, _D), jnp.float32),
        jax.ShapeDtypeStruct((16, _D), jnp.float32),
    ),
    scratch_types=[
        pltpu.VMEM((_B,), jnp.int32),
        pltpu.VMEM((_B + 16,), jnp.int32),
        pltpu.VMEM((_D, _SUP), jnp.float32),
        pltpu.VMEM((_D, _SUP), jnp.float32),
        pltpu.VMEM((4112,), jnp.int32),
        pltpu.VMEM((4112,), jnp.int32),
        pltpu.VMEM((_RING * 16, _D), jnp.float32),
        pltpu.SemaphoreType.DMA,
        pltpu.SemaphoreType.DMA,
        pltpu.SemaphoreType.DMA,
        pltpu.SemaphoreType.DMA,
    ],
    compiler_params=pltpu.CompilerParams(needs_layout_passes=False),
)
def _gather(tab_t, idx_hbm, out, out2, el_v, slot_list, blk0, blk1,
            hslot, hpos, rowbuf, isem, sem0, sem1, wsem):
    wid = lax.axis_index("s") * _NC + lax.axis_index("c")
    lane = lax.iota(jnp.int32, 16)

    pltpu.async_copy(idx_hbm, el_v, isem).wait()

    def scan(p, cnt):
        vec = el_v[pl.ds(p * 16, 16)]
        m = (vec >> 15) == wid
        plsc.store_compressed(slot_list.at[pl.ds(cnt, 16)], p * 16 + lane,
                              mask=m)
        return cnt + plsc.all_reduce_population_count(m)[0]

    cnt = lax.fori_loop(0, _B // 16, scan, 0)

    def fire(s, blk, sem):
        @pl.when(wid * _NSUP + s <= _LAST_FULL)
        def _():
            base = pl.multiple_of((wid * _NSUP + s) * _SUP, 128)
            pltpu.async_copy(tab_t.at[:, pl.ds(base, _SUP)], blk, sem)

    def drain_blk(s, blk, sem):
        @pl.when(wid * _NSUP + s <= _LAST_FULL)
        def _():
            pltpu.make_async_copy(tab_t.at[:, pl.ds(0, _SUP)], blk, sem).wait()

    def extract(s, blk, width):
        gsup = wid * _NSUP + s

        def filt(q, hcnt):
            slotv = slot_list[pl.ds(q * 16, 16)]
            ok = (q * 16 + lane) < cnt
            rv = plsc.load_gather(el_v, [slotv], mask=ok)
            m2 = ok & ((rv >> 10) == gsup)
            plsc.store_compressed(hslot.at[pl.ds(hcnt, 16)], slotv, mask=m2)
            plsc.store_compressed(hpos.at[pl.ds(hcnt, 16)], rv & (_SUP - 1),
                                  mask=m2)
            return hcnt + plsc.all_reduce_population_count(m2)[0]

        hcnt = lax.fori_loop(0, (cnt + 15) >> 4, filt, 0)
        nh = (hcnt + 15) >> 4

        def group(k, carry):
            @pl.when(k >= _RING)
            def _():
                pltpu.make_async_copy(
                    rowbuf.at[pl.ds(0, 16)], out2, wsem
                ).wait()

            hs = hslot[pl.ds(k * 16, 16)]
            hp = hpos[pl.ds(k * 16, 16)]
            hv = (k * 16 + lane) < hcnt
            rbase = (k & (_RING - 1)) * 16
            for c in range(_D):
                vals = plsc.load_gather(
                    blk, [jnp.full((16,), c, jnp.int32), hp], mask=hv
                )
                plsc.store_scatter(
                    rowbuf,
                    [rbase + lane, jnp.full((16,), c, jnp.int32)],
                    vals,
                    mask=hv,
                )
            for j in range(16):
                slot = hs[j]
                valid_j = (k * 16 + j) < hcnt

                @pl.when(valid_j)
                def _():
                    pltpu.async_copy(
                        rowbuf.at[pl.ds(rbase + j, 1)],
                        out.at[pl.ds(slot, 1)],
                        wsem,
                    )

                @pl.when(jnp.logical_not(valid_j))
                def _():
                    pltpu.async_copy(
                        rowbuf.at[pl.ds(rbase + j, 1)],
                        out2.at[pl.ds(j, 1)],
                        wsem,
                    )
            return carry

        lax.fori_loop(0, nh, group, 0)

        def final_drain(k, carry):
            pltpu.make_async_copy(rowbuf.at[pl.ds(0, 16)], out2, wsem).wait()
            return carry

        lax.fori_loop(0, jnp.minimum(nh, _RING), final_drain, 0)
        del width

    fire(0, blk0, sem0)

    def pair(p, carry):
        s0 = p * 2
        fire(s0 + 1, blk1, sem1)
        drain_blk(s0, blk0, sem0)

        @pl.when(wid * _NSUP + s0 <= _LAST_FULL)
        def _():
            extract(s0, blk0, _SUP)

        @pl.when(p < _NSUP // 2 - 1)
        def _():
            fire(s0 + 2, blk0, sem0)

        drain_blk(s0 + 1, blk1, sem1)

        @pl.when(wid * _NSUP + s0 + 1 <= _LAST_FULL)
        def _():
            extract(s0 + 1, blk1, _SUP)

        return carry

    lax.fori_loop(0, _NSUP // 2, pair, 0)

    @pl.when(wid == 30)
    def _():
        tail_base = pl.multiple_of(_TAIL_BASE + (wid - 30) * 128, 128)
        pltpu.async_copy(
            tab_t.at[:, pl.ds(tail_base, _TAIL)],
            blk0.at[:, pl.ds(0, _TAIL)],
            sem0,
        )
        pltpu.make_async_copy(
            tab_t.at[:, pl.ds(0, _TAIL)], blk0.at[:, pl.ds(0, _TAIL)], sem0
        ).wait()
        extract(16, blk0, _TAIL)


def kernel(x, el):
    out, _ = _gather(x.T, el.astype(jnp.int32))
    return out


# speedup vs baseline: 1.1897x; 1.1897x over previous
import functools

import jax
import jax.numpy as jnp
from jax import lax
from jax.experimental import pallas as pl
from jax.experimental.pallas import tpu as pltpu
from jax.experimental.pallas import tpu_sc as plsc

_NC = 2
_NS = 16
_NW = _NC * _NS
_B = 16384
_D = 32
_BPW = _B // _NW
_G = 8
_NG = _BPW // _G

_mesh = plsc.VectorSubcoreMesh(core_axis_name="c", subcore_axis_name="s")


@functools.partial(
    pl.kernel,
    mesh=_mesh,
    out_type=jax.ShapeDtypeStruct((_D, _B), jnp.float32),
    scratch_types=[
        pltpu.VMEM((_BPW,), jnp.int32),
        pltpu.VMEM((_D, _G * 128), jnp.float32),
        pltpu.VMEM((_D, _G * 128), jnp.float32),
        pltpu.VMEM((_D, _BPW), jnp.float32),
        pltpu.SemaphoreType.DMA,
        pltpu.SemaphoreType.DMA,
        pltpu.SemaphoreType.DMA,
    ],
    compiler_params=pltpu.CompilerParams(needs_layout_passes=False),
)
def _gather(tab_t, idx_hbm, out_t, idx_v, blk0, blk1, cols_v, isem, sem0, sem1):
    wid = lax.axis_index("s") * _NC + lax.axis_index("c")
    base = wid * _BPW
    pltpu.async_copy(idx_hbm.at[pl.ds(base, _BPW)], idx_v, isem).wait()

    lane = lax.iota(jnp.int32, 16)

    def fire(vec16, j0, blk, sem):
        for j in range(_G):
            rb = pl.multiple_of((vec16[j0 + j] >> 7) * 128, 128)
            pltpu.async_copy(
                tab_t.at[:, pl.ds(rb, 128)],
                blk.at[:, pl.ds(j * 128, 128)],
                sem,
            )

    def drain(blk, sem):
        pltpu.make_async_copy(tab_t.at[:, pl.ds(0, _G * 128)], blk, sem).wait()

    def extract(g, vec16, j0, blk):
        lvec = vec16 & 127
        for j in range(_G):
            pos = jnp.full((16,), j * 128 + lvec[j0 + j], jnp.int32)
            slot = jnp.full((16,), g * _G + j, jnp.int32)
            lo = plsc.load_gather(blk, [lane, pos])
            hi = plsc.load_gather(blk, [lane + 16, pos])
            plsc.store_scatter(cols_v, [lane, slot], lo)
            plsc.store_scatter(cols_v, [lane + 16, slot], hi)

    fire(idx_v[pl.ds(0, 16)], 0, blk0, sem0)

    def pair(p, carry):
        g0 = p * 2
        vec16 = idx_v[pl.ds(p * 16, 16)]
        fire(vec16, _G, blk1, sem1)
        drain(blk0, sem0)
        extract(g0, vec16, 0, blk0)

        @pl.when(p < _NG // 2 - 1)
        def _():
            fire(idx_v[pl.ds(p * 16 + 16, 16)], 0, blk0, sem0)

        drain(blk1, sem1)
        extract(g0 + 1, vec16, _G, blk1)
        return carry

    lax.fori_loop(0, _NG // 2, pair, 0)
    pltpu.sync_copy(cols_v, out_t.at[:, pl.ds(base, _BPW)])


def kernel(x, el):
    return _gather(x.T, el.astype(jnp.int32)).T
